# Initial kernel scaffold; baseline (speedup 1.0000x reference)
import jax
import jax.numpy as jnp
from jax.experimental import pallas as pl


def kernel(x, edge_index, W1_0, b1_0, W2_0, b2_0, Wr_0, br_0, W1_1, b1_1, W2_1, b2_1, Wr_1, br_1):
    raise NotImplementedError("write your pallas kernel here")


# speedup vs baseline: 2.2501x; 2.2501x over previous
import functools

import jax
import jax.numpy as jnp
from jax import lax
from jax.experimental import pallas as pl
from jax.experimental.pallas import tpu as pltpu
from jax.experimental.pallas import tpu_sc as plsc

NE = 8
NC = 2
NS = 16
LB = 128


def _make_seg(n, n_batches, n_slices, with_deg):
    acc_rows = ((n + 1) + NS * 8 - 1) // (NS * 8) * (NS * 8)
    zr = acc_rows // NS
    rps = n_batches // NS
    rps2 = n_batches // (NC * NS)
    spc = n_slices // NC
    mesh = plsc.VectorSubcoreMesh(core_axis_name="c", subcore_axis_name="s")

    out_type = [jax.ShapeDtypeStruct((n_slices, acc_rows, LB), jnp.float32)]
    scratch = [
        pltpu.VMEM_SHARED((acc_rows, LB), jnp.float32),
        pltpu.VMEM((LB, LB), jnp.float32),
        pltpu.VMEM((LB,), jnp.int32),
        pltpu.VMEM((LB,), jnp.int32),
        pltpu.SemaphoreType.DMA,
    ]
    if with_deg:
        out_type.append(
            jax.ShapeDtypeStruct((NC, acc_rows, LB), jnp.float32))
        scratch.append(pltpu.VMEM((LB, LB), jnp.float32))

    def body(*refs):
        if with_deg:
            (xflat, srcoff, d2d, zacc, ones_in,
             agg_out, deg_out, acc, rows, sidx, didx, sem, onesb) = refs
        else:
            (xflat, srcoff, d2d, zacc,
             agg_out, acc, rows, sidx, didx, sem) = refs
        c = lax.axis_index("c")
        s = lax.axis_index("s")
        pltpu.sync_copy(zacc, acc.at[pl.ds(s * zr, zr)])
        if with_deg:
            pltpu.sync_copy(ones_in, onesb)
        plsc.subcore_barrier()
        for phase in range(spc):
            sl = c * spc + phase

            @pl.loop(0, rps)
            def _(jj):
                row = s * rps + jj
                pltpu.sync_copy(srcoff.at[sl, row], sidx)
                pltpu.sync_copy(d2d.at[row], didx)
                pltpu.async_copy(xflat.at[sidx], rows, sem).wait()
                pltpu.sync_copy(rows, acc.at[didx], add=True)

            plsc.subcore_barrier()
            pltpu.sync_copy(acc.at[pl.ds(s * zr, zr)],
                            agg_out.at[sl, pl.ds(s * zr, zr)])
            if phase + 1 < spc or with_deg:
                pltpu.sync_copy(zacc, acc.at[pl.ds(s * zr, zr)])
                plsc.subcore_barrier()
        if with_deg:
            @pl.loop(0, rps2)
            def _(jj):
                row = (c * NS + s) * rps2 + jj
                pltpu.sync_copy(d2d.at[row], didx)
                pltpu.sync_copy(onesb, acc.at[didx], add=True)

            plsc.subcore_barrier()
            pltpu.sync_copy(acc.at[pl.ds(s * zr, zr)],
                            deg_out.at[c, pl.ds(s * zr, zr)])

    return pl.kernel(body, out_type=tuple(out_type), mesh=mesh,
                     scratch_types=scratch)


def _moe_body(h_ref, agg_ref, deg_ref, wr_ref, br_ref, w1_ref, b1_ref,
              w2_ref, b2_ref, out_ref, m_s, w_s, *, n_slices, relu_out):
    e = pl.program_id(1)

    @pl.when(e == 0)
    def _():
        degb = deg_ref[...]
        recip = 1.0 / jnp.maximum(degb, 1.0)
        aggc = jnp.concatenate([agg_ref[s] for s in range(n_slices)], axis=-1)
        m_s[...] = h_ref[...] + aggc * recip
        feat = jnp.log1p(degb)
        logits = feat * wr_ref[...] + br_ref[...]
        it = lax.broadcasted_iota(jnp.int32, logits.shape, 1)
        v1 = jnp.max(logits, axis=-1, keepdims=True)
        i1 = jnp.min(jnp.where(logits == v1, it, NE), axis=-1, keepdims=True)
        oh1 = it == i1
        l2 = jnp.where(oh1, -1e30, logits)
        v2 = jnp.max(l2, axis=-1, keepdims=True)
        i2 = jnp.min(jnp.where(l2 == v2, it, NE), axis=-1, keepdims=True)
        oh2 = it == i2
        t = jnp.exp(v2 - v1)
        g1 = 1.0 / (1.0 + t)
        g2 = t / (1.0 + t)
        w_s[...] = (jnp.where(oh1, g1, 0.0) + jnp.where(oh2, g2, 0.0))

    hid = jnp.maximum(
        jnp.dot(m_s[...], w1_ref[0], preferred_element_type=jnp.float32)
        + b1_ref[0], 0.0)
    o = jnp.dot(hid, w2_ref[0], preferred_element_type=jnp.float32) \
        + b2_ref[0]
    lane = lax.broadcasted_iota(jnp.int32, w_s.shape, 1)
    wcol = jnp.sum(jnp.where(lane == e, w_s[...], 0.0), axis=-1,
                   keepdims=True)
    contrib = wcol * o

    @pl.when(e == 0)
    def _():
        out_ref[...] = contrib

    @pl.when(e > 0)
    def _():
        out_ref[...] = out_ref[...] + contrib

    if relu_out:
        @pl.when(e == NE - 1)
        def _():
            out_ref[...] = jnp.maximum(out_ref[...], 0.0)


def _make_moe(n, d_in, d_h, d_out, n_slices, relu_out, block=2000):
    nb = n // block
    grid = (nb, NE)
    body = functools.partial(_moe_body, n_slices=n_slices, relu_out=relu_out)
    return pl.pallas_call(
        body,
        grid=grid,
        in_specs=[
            pl.BlockSpec((block, d_in), lambda i, e: (i, 0)),
            pl.BlockSpec((n_slices, block, LB), lambda i, e: (0, i, 0)),
            pl.BlockSpec((block, 1), lambda i, e: (i, 0)),
            pl.BlockSpec((1, NE), lambda i, e: (0, 0)),
            pl.BlockSpec((1, NE), lambda i, e: (0, 0)),
            pl.BlockSpec((1, d_in, d_h), lambda i, e: (e, 0, 0)),
            pl.BlockSpec((1, 1, d_h), lambda i, e: (e, 0, 0)),
            pl.BlockSpec((1, d_h, d_out), lambda i, e: (e, 0, 0)),
            pl.BlockSpec((1, 1, d_out), lambda i, e: (e, 0, 0)),
        ],
        out_specs=pl.BlockSpec((block, d_out), lambda i, e: (i, 0)),
        out_shape=jax.ShapeDtypeStruct((n, d_out), jnp.float32),
        scratch_shapes=[
            pltpu.VMEM((block, d_in), jnp.float32),
            pltpu.VMEM((block, NE), jnp.float32),
        ],
        compiler_params=pltpu.CompilerParams(
            dimension_semantics=("parallel", "arbitrary")),
    )


def kernel(x, edge_index, W1_0, b1_0, W2_0, b2_0, Wr_0, br_0,
           W1_1, b1_1, W2_1, b2_1, Wr_1, br_1):
    n, d_in = x.shape
    e_num = edge_index.shape[1]
    d_h = W1_0.shape[2]
    d_out = W2_1.shape[2]
    s0 = d_in // LB
    s1 = d_h // LB

    src = edge_index[0]
    dst = edge_index[1]
    ep = -(-e_num // (NC * NS * LB)) * (NC * NS * LB)
    srcp = jnp.pad(src, (0, ep - e_num))
    dstp = jnp.pad(dst, (0, ep - e_num), constant_values=n)
    nb = ep // LB
    s2d = srcp.reshape(nb, LB)
    d2d = dstp.reshape(nb, LB)
    srcoff0 = jnp.stack([s2d + t * n for t in range(s0)])
    srcoff1 = jnp.stack([s2d + t * n for t in range(s1)])

    acc_rows = ((n + 1) + NS * 8 - 1) // (NS * 8) * (NS * 8)
    zr = acc_rows // NS
    zacc = jnp.zeros((zr, LB), jnp.float32)
    ones = jnp.ones((LB, LB), jnp.float32)

    xflat = jnp.stack([x[:, t * LB:(t + 1) * LB] for t in range(s0)])
    xflat = xflat.reshape(s0 * n, LB)
    seg0 = _make_seg(n, nb, s0, with_deg=True)
    agg0_st, degw = seg0(xflat, srcoff0, d2d, zacc, ones)
    agg0 = agg0_st[:, :n, :]
    deg = (degw[0, :n, 0] + degw[1, :n, 0])[:, None]

    wr0 = Wr_0.reshape(1, NE)
    br0 = br_0.reshape(1, NE)
    moe0 = _make_moe(n, d_in, d_h, d_h, s0, relu_out=True)
    h = moe0(x, agg0, deg, wr0, br0, W1_0, b1_0.reshape(NE, 1, d_h),
             W2_0, b2_0.reshape(NE, 1, d_h))

    hflat = jnp.stack([h[:, t * LB:(t + 1) * LB] for t in range(s1)])
    hflat = hflat.reshape(s1 * n, LB)
    seg1 = _make_seg(n, nb, s1, with_deg=False)
    (agg1_st,) = seg1(hflat, srcoff1, d2d, zacc)
    agg1 = agg1_st[:, :n, :]

    wr1 = Wr_1.reshape(1, NE)
    br1 = br_1.reshape(1, NE)
    moe1 = _make_moe(n, d_h, d_h, d_out, s1, relu_out=False)
    out = moe1(h, agg1, deg, wr1, br1, W1_1, b1_1.reshape(NE, 1, d_h),
               W2_1, b2_1.reshape(NE, 1, d_out))
    return out


# speedup vs baseline: 3.8087x; 1.6927x over previous
import functools

import jax
import jax.numpy as jnp
from jax import lax
from jax.experimental import pallas as pl
from jax.experimental.pallas import tpu as pltpu
from jax.experimental.pallas import tpu_sc as plsc

NE = 8
NC = 2
NS = 16
LB = 128


def _make_seg(n, n_batches, n_slices, with_deg):
    acc_rows = ((n + 1) + NS * 8 - 1) // (NS * 8) * (NS * 8)
    zr = acc_rows // NS
    rps = n_batches // NS
    rps2 = n_batches // (NC * NS)
    spc = n_slices // NC
    mesh = plsc.VectorSubcoreMesh(core_axis_name="c", subcore_axis_name="s")

    out_type = [jax.ShapeDtypeStruct((n_slices, acc_rows, LB), jnp.float32)]
    scratch = [
        pltpu.VMEM_SHARED((acc_rows, LB), jnp.float32),
        pltpu.VMEM((2, LB, LB), jnp.float32),
        pltpu.VMEM((4, LB), jnp.int32),
        pltpu.VMEM((4, LB), jnp.int32),
        pltpu.SemaphoreType.DMA,
        pltpu.SemaphoreType.DMA,
        pltpu.SemaphoreType.DMA,
    ]
    if with_deg:
        out_type.append(
            jax.ShapeDtypeStruct((NC, acc_rows, LB), jnp.float32))

    def body(*refs):
        if with_deg:
            (xflat, srcoff, d2d, zacc, ones_in,
             agg_out, deg_out, acc, rows, sidx, didx, isem, gsem,
             ssem) = refs
        else:
            (xflat, srcoff, d2d, zacc,
             agg_out, acc, rows, sidx, didx, isem, gsem, ssem) = refs
        c = lax.axis_index("c")
        s = lax.axis_index("s")

        def fire_idx(sl, k):
            slot = lax.rem(k, 4)
            pltpu.async_copy(srcoff.at[sl, s * rps + k], sidx.at[slot], isem)
            pltpu.async_copy(d2d.at[s * rps + k], didx.at[slot], isem)

        def drain_idx():
            pltpu.make_async_copy(d2d.at[0], didx.at[0], isem).wait()
            pltpu.make_async_copy(d2d.at[0], didx.at[0], isem).wait()

        def fire_gather(k):
            pltpu.async_copy(xflat.at[sidx.at[lax.rem(k, 4)]],
                             rows.at[lax.rem(k, 2)], gsem)

        def wait_gather():
            pltpu.make_async_copy(xflat.at[sidx.at[0]], rows.at[0],
                                  gsem).wait()

        pltpu.sync_copy(zacc, acc.at[pl.ds(s * zr, zr)])
        plsc.subcore_barrier()
        for phase in range(spc):
            sl = c * spc + phase
            for k in range(3):
                fire_idx(sl, k)
            drain_idx()
            fire_gather(0)
            drain_idx()
            fire_gather(1)

            @pl.loop(0, rps)
            def _(j):
                @pl.when(j + 3 < rps)
                def _():
                    fire_idx(sl, j + 3)

                wait_gather()
                pltpu.sync_copy(rows.at[lax.rem(j, 2)],
                                acc.at[didx.at[lax.rem(j, 4)]], add=True)

                @pl.when(j + 2 < rps)
                def _():
                    drain_idx()
                    fire_gather(j + 2)

            plsc.subcore_barrier()
            pltpu.sync_copy(acc.at[pl.ds(s * zr, zr)],
                            agg_out.at[sl, pl.ds(s * zr, zr)])
            if phase + 1 < spc or with_deg:
                pltpu.sync_copy(zacc, acc.at[pl.ds(s * zr, zr)])
                plsc.subcore_barrier()
        if with_deg:
            base = (c * NS + s) * rps2
            pltpu.sync_copy(ones_in, rows.at[0])

            def fire_didx(k):
                pltpu.async_copy(d2d.at[base + k], didx.at[lax.rem(k, 4)],
                                 isem)

            def drain_didx():
                pltpu.make_async_copy(d2d.at[0], didx.at[0], isem).wait()

            def drain_scat():
                pltpu.make_async_copy(rows.at[0], acc.at[didx.at[0]],
                                      ssem).wait()

            fire_didx(0)
            fire_didx(1)

            @pl.loop(0, rps2)
            def _(j):
                @pl.when(j >= 2)
                def _():
                    drain_scat()

                @pl.when(j + 2 < rps2)
                def _():
                    fire_didx(j + 2)

                drain_didx()
                pltpu.async_copy(rows.at[0], acc.at[didx.at[lax.rem(j, 4)]],
                                 ssem, add=True)

            drain_scat()
            drain_scat()
            plsc.subcore_barrier()
            pltpu.sync_copy(acc.at[pl.ds(s * zr, zr)],
                            deg_out.at[c, pl.ds(s * zr, zr)])

    return pl.kernel(body, out_type=tuple(out_type), mesh=mesh,
                     scratch_types=scratch)


def _moe_body(h_ref, agg_ref, deg_ref, wr_ref, br_ref, w1_ref, b1_ref,
              w2_ref, b2_ref, out_ref, m_s, w_s, *, n_slices, relu_out):
    e = pl.program_id(1)

    @pl.when(e == 0)
    def _():
        degb = deg_ref[...]
        recip = 1.0 / jnp.maximum(degb, 1.0)
        aggc = jnp.concatenate([agg_ref[s] for s in range(n_slices)], axis=-1)
        m_s[...] = h_ref[...] + aggc * recip
        feat = jnp.log1p(degb)
        logits = feat * wr_ref[...] + br_ref[...]
        it = lax.broadcasted_iota(jnp.int32, logits.shape, 1)
        v1 = jnp.max(logits, axis=-1, keepdims=True)
        i1 = jnp.min(jnp.where(logits == v1, it, NE), axis=-1, keepdims=True)
        oh1 = it == i1
        l2 = jnp.where(oh1, -1e30, logits)
        v2 = jnp.max(l2, axis=-1, keepdims=True)
        i2 = jnp.min(jnp.where(l2 == v2, it, NE), axis=-1, keepdims=True)
        oh2 = it == i2
        t = jnp.exp(v2 - v1)
        g1 = 1.0 / (1.0 + t)
        g2 = t / (1.0 + t)
        w_s[...] = (jnp.where(oh1, g1, 0.0) + jnp.where(oh2, g2, 0.0))

    hid = jnp.maximum(
        jnp.dot(m_s[...], w1_ref[0], preferred_element_type=jnp.float32)
        + b1_ref[0], 0.0)
    o = jnp.dot(hid, w2_ref[0], preferred_element_type=jnp.float32) \
        + b2_ref[0]
    lane = lax.broadcasted_iota(jnp.int32, w_s.shape, 1)
    wcol = jnp.sum(jnp.where(lane == e, w_s[...], 0.0), axis=-1,
                   keepdims=True)
    contrib = wcol * o

    @pl.when(e == 0)
    def _():
        out_ref[...] = contrib

    @pl.when(e > 0)
    def _():
        out_ref[...] = out_ref[...] + contrib

    if relu_out:
        @pl.when(e == NE - 1)
        def _():
            out_ref[...] = jnp.maximum(out_ref[...], 0.0)


def _make_moe(n, d_in, d_h, d_out, n_slices, relu_out, block=2000):
    nb = n // block
    grid = (nb, NE)
    body = functools.partial(_moe_body, n_slices=n_slices, relu_out=relu_out)
    return pl.pallas_call(
        body,
        grid=grid,
        in_specs=[
            pl.BlockSpec((block, d_in), lambda i, e: (i, 0)),
            pl.BlockSpec((n_slices, block, LB), lambda i, e: (0, i, 0)),
            pl.BlockSpec((block, 1), lambda i, e: (i, 0)),
            pl.BlockSpec((1, NE), lambda i, e: (0, 0)),
            pl.BlockSpec((1, NE), lambda i, e: (0, 0)),
            pl.BlockSpec((1, d_in, d_h), lambda i, e: (e, 0, 0)),
            pl.BlockSpec((1, 1, d_h), lambda i, e: (e, 0, 0)),
            pl.BlockSpec((1, d_h, d_out), lambda i, e: (e, 0, 0)),
            pl.BlockSpec((1, 1, d_out), lambda i, e: (e, 0, 0)),
        ],
        out_specs=pl.BlockSpec((block, d_out), lambda i, e: (i, 0)),
        out_shape=jax.ShapeDtypeStruct((n, d_out), jnp.float32),
        scratch_shapes=[
            pltpu.VMEM((block, d_in), jnp.float32),
            pltpu.VMEM((block, NE), jnp.float32),
        ],
        compiler_params=pltpu.CompilerParams(
            dimension_semantics=("parallel", "arbitrary")),
    )


def kernel(x, edge_index, W1_0, b1_0, W2_0, b2_0, Wr_0, br_0,
           W1_1, b1_1, W2_1, b2_1, Wr_1, br_1):
    n, d_in = x.shape
    e_num = edge_index.shape[1]
    d_h = W1_0.shape[2]
    d_out = W2_1.shape[2]
    s0 = d_in // LB
    s1 = d_h // LB

    src = edge_index[0]
    dst = edge_index[1]
    ep = -(-e_num // (NC * NS * LB)) * (NC * NS * LB)
    srcp = jnp.pad(src, (0, ep - e_num))
    dstp = jnp.pad(dst, (0, ep - e_num), constant_values=n)
    nb = ep // LB
    s2d = srcp.reshape(nb, LB)
    d2d = dstp.reshape(nb, LB)
    srcoff0 = jnp.stack([s2d + t * n for t in range(s0)])
    srcoff1 = jnp.stack([s2d + t * n for t in range(s1)])

    acc_rows = ((n + 1) + NS * 8 - 1) // (NS * 8) * (NS * 8)
    zr = acc_rows // NS
    zacc = jnp.zeros((zr, LB), jnp.float32)
    ones = jnp.ones((LB, LB), jnp.float32)

    xflat = jnp.stack([x[:, t * LB:(t + 1) * LB] for t in range(s0)])
    xflat = xflat.reshape(s0 * n, LB)
    seg0 = _make_seg(n, nb, s0, with_deg=True)
    agg0_st, degw = seg0(xflat, srcoff0, d2d, zacc, ones)
    agg0 = agg0_st[:, :n, :]
    deg = (degw[0, :n, 0] + degw[1, :n, 0])[:, None]

    wr0 = Wr_0.reshape(1, NE)
    br0 = br_0.reshape(1, NE)
    moe0 = _make_moe(n, d_in, d_h, d_h, s0, relu_out=True)
    h = moe0(x, agg0, deg, wr0, br0, W1_0, b1_0.reshape(NE, 1, d_h),
             W2_0, b2_0.reshape(NE, 1, d_h))

    hflat = jnp.stack([h[:, t * LB:(t + 1) * LB] for t in range(s1)])
    hflat = hflat.reshape(s1 * n, LB)
    seg1 = _make_seg(n, nb, s1, with_deg=False)
    (agg1_st,) = seg1(hflat, srcoff1, d2d, zacc)
    agg1 = agg1_st[:, :n, :]

    wr1 = Wr_1.reshape(1, NE)
    br1 = br_1.reshape(1, NE)
    moe1 = _make_moe(n, d_h, d_h, d_out, s1, relu_out=False)
    out = moe1(h, agg1, deg, wr1, br1, W1_1, b1_1.reshape(NE, 1, d_h),
               W2_1, b2_1.reshape(NE, 1, d_out))
    return out
